# Initial kernel scaffold; baseline (speedup 1.0000x reference)
import jax
import jax.numpy as jnp
from jax.experimental import pallas as pl


def kernel(x, edge_index, batch, W1l, b1l, W1r, b1r, W2l, b2l, W2r, b2r, Wlin, blin):
    raise NotImplementedError("write your pallas kernel here")


# speedup vs baseline: 6.0237x; 6.0237x over previous
import functools

import jax
import jax.numpy as jnp
from jax import lax
from jax.experimental import pallas as pl
from jax.experimental.pallas import tpu as pltpu
from jax.experimental.pallas import tpu_sc as plsc

N = 10000
E = 320000
D = 128
G = 64

NC = 2
NS = 16
NW = NC * NS
EPW = E // NW
CH = 80
NFULL = EPW // CH
NPAIR = NFULL // 2
RPT = 624
RPT_LAST = N - (NS - 1) * RPT
NZF = RPT // CH
REM = RPT - NZF * CH

R = 400
NB = N // R

_MESH = plsc.VectorSubcoreMesh(core_axis_name="c", subcore_axis_name="s")


def _zero_acc(sid, r0, stage, acc):
    @pl.when(sid < NS - 1)
    def _():
        @pl.loop(0, NZF)
        def zc(j):
            pltpu.sync_copy(stage, acc.at[pl.ds(r0 + j * CH, CH)])
        pltpu.sync_copy(stage.at[pl.ds(0, REM)],
                        acc.at[pl.ds(r0 + NZF * CH, REM)])

    @pl.when(sid == NS - 1)
    def _():
        rl = (NS - 1) * RPT

        @pl.loop(0, RPT_LAST // CH)
        def zc(j):
            pltpu.sync_copy(stage, acc.at[pl.ds(rl + j * CH, CH)])


def _write_acc(cid, sid, r0, acc, stage, out_hbm):
    @pl.when(sid < NS - 1)
    def _():
        @pl.loop(0, NZF)
        def wc(j):
            pltpu.sync_copy(acc.at[pl.ds(r0 + j * CH, CH)], stage)
            pltpu.sync_copy(stage,
                            out_hbm.at[pl.ds(cid * N + r0 + j * CH, CH)])
        pltpu.sync_copy(acc.at[pl.ds(r0 + NZF * CH, REM)],
                        stage.at[pl.ds(0, REM)])
        pltpu.sync_copy(stage.at[pl.ds(0, REM)],
                        out_hbm.at[pl.ds(cid * N + r0 + NZF * CH, REM)])

    @pl.when(sid == NS - 1)
    def _():
        rl = (NS - 1) * RPT

        @pl.loop(0, RPT_LAST // CH)
        def wc(j):
            pltpu.sync_copy(acc.at[pl.ds(rl + j * CH, CH)], stage)
            pltpu.sync_copy(stage,
                            out_hbm.at[pl.ds(cid * N + rl + j * CH, CH)])


@functools.partial(
    pl.kernel, mesh=_MESH,
    out_type=[jax.ShapeDtypeStruct((NC * N, D), jnp.float32)],
    scratch_types=[
        pltpu.VMEM((CH,), jnp.int32),
        pltpu.VMEM((CH,), jnp.int32),
        pltpu.VMEM((CH,), jnp.int32),
        pltpu.VMEM((CH,), jnp.int32),
        pltpu.VMEM((CH, D), jnp.float32),
        pltpu.VMEM((CH, D), jnp.float32),
        pltpu.VMEM_SHARED((N, D), jnp.float32),
        pltpu.SemaphoreType.DMA,
        pltpu.SemaphoreType.DMA,
    ])
def _agg(h_hbm, src_hbm, dst_hbm, z2d_hbm, psum_hbm,
         s0, s1, d0, d1, rows0, rows1, acc, sem0, sem1):
    cid = lax.axis_index("c")
    sid = lax.axis_index("s")
    wid = cid * NS + sid
    base = wid * EPW
    r0 = sid * RPT

    pltpu.sync_copy(z2d_hbm, rows0)
    _zero_acc(sid, r0, rows0, acc)
    plsc.subcore_barrier()

    pltpu.sync_copy(src_hbm.at[pl.ds(base, CH)], s0)
    pltpu.sync_copy(dst_hbm.at[pl.ds(base, CH)], d0)
    pltpu.async_copy(h_hbm.at[s0], rows0, sem0)
    pltpu.sync_copy(src_hbm.at[pl.ds(base + CH, CH)], s1)
    pltpu.sync_copy(dst_hbm.at[pl.ds(base + CH, CH)], d1)
    pltpu.async_copy(h_hbm.at[s1], rows1, sem1)

    @pl.loop(0, NPAIR)
    def pair(i):
        c0 = 2 * i
        pltpu.make_async_copy(h_hbm.at[s0], rows0, sem0).wait()
        pltpu.sync_copy(rows0, acc.at[d0], add=True)
        pltpu.make_async_copy(h_hbm.at[s1], rows1, sem1).wait()
        pltpu.sync_copy(rows1, acc.at[d1], add=True)

        @pl.when(c0 + 2 < NFULL)
        def _():
            off = base + (c0 + 2) * CH
            pltpu.sync_copy(src_hbm.at[pl.ds(off, CH)], s0)
            pltpu.sync_copy(dst_hbm.at[pl.ds(off, CH)], d0)
            pltpu.async_copy(h_hbm.at[s0], rows0, sem0)

        @pl.when(c0 + 3 < NFULL)
        def _():
            off = base + (c0 + 3) * CH
            pltpu.sync_copy(src_hbm.at[pl.ds(off, CH)], s1)
            pltpu.sync_copy(dst_hbm.at[pl.ds(off, CH)], d1)
            pltpu.async_copy(h_hbm.at[s1], rows1, sem1)

    pltpu.make_async_copy(h_hbm.at[s0], rows0, sem0).wait()
    pltpu.sync_copy(rows0, acc.at[d0], add=True)

    plsc.subcore_barrier()
    _write_acc(cid, sid, r0, acc, rows0, psum_hbm)


@functools.partial(
    pl.kernel, mesh=_MESH,
    out_type=[jax.ShapeDtypeStruct((NC * N, D), jnp.float32)],
    scratch_types=[
        pltpu.VMEM((CH,), jnp.int32),
        pltpu.VMEM((CH, D), jnp.float32),
        pltpu.VMEM((CH, D), jnp.float32),
        pltpu.VMEM_SHARED((N, D), jnp.float32),
    ])
def _deg(dst_hbm, z2d_hbm, o2d_hbm, cnt_hbm, d0, stage, ones_v, acc):
    cid = lax.axis_index("c")
    sid = lax.axis_index("s")
    wid = cid * NS + sid
    base = wid * EPW
    r0 = sid * RPT

    pltpu.sync_copy(z2d_hbm, stage)
    pltpu.sync_copy(o2d_hbm, ones_v)
    _zero_acc(sid, r0, stage, acc)
    plsc.subcore_barrier()

    @pl.loop(0, NFULL)
    def chunk(i):
        pltpu.sync_copy(dst_hbm.at[pl.ds(base + i * CH, CH)], d0)
        pltpu.sync_copy(ones_v, acc.at[d0], add=True)

    plsc.subcore_barrier()
    _write_acc(cid, sid, r0, acc, stage, cnt_hbm)


def _dense1_body(p0, p1, c0, c1, x, wl, wr, b, h_out, rc_out):
    cnt = c0[:, 0:1] + c1[:, 0:1]
    rc = 1.0 / jnp.maximum(cnt, 1.0)
    mean = (p0[...] + p1[...]) * rc
    h = (jnp.dot(mean, wl[...], preferred_element_type=jnp.float32)
         + jnp.dot(x[...], wr[...], preferred_element_type=jnp.float32)
         + b[...])
    h_out[...] = jnp.maximum(h, 0.0)
    rc_out[...] = rc


_dense1 = pl.pallas_call(
    _dense1_body,
    grid=(NB,),
    in_specs=[
        pl.BlockSpec((R, D), lambda i: (i, 0)),
        pl.BlockSpec((R, D), lambda i: (i + NB, 0)),
        pl.BlockSpec((R, D), lambda i: (i, 0)),
        pl.BlockSpec((R, D), lambda i: (i + NB, 0)),
        pl.BlockSpec((R, D), lambda i: (i, 0)),
        pl.BlockSpec((D, D), lambda i: (0, 0)),
        pl.BlockSpec((D, D), lambda i: (0, 0)),
        pl.BlockSpec((1, D), lambda i: (0, 0)),
    ],
    out_specs=[pl.BlockSpec((R, D), lambda i: (i, 0)),
               pl.BlockSpec((R, 1), lambda i: (i, 0))],
    out_shape=[jax.ShapeDtypeStruct((N, D), jnp.float32),
               jax.ShapeDtypeStruct((N, 1), jnp.float32)],
)


def _dense2_body(p0, p1, rc, h1, wl, wr, b, bt, wlin, blin, out,
                 gsum, gcnt):
    i = pl.program_id(0)
    mean = (p0[...] + p1[...]) * rc[...]
    h = (jnp.dot(mean, wl[...], preferred_element_type=jnp.float32)
         + jnp.dot(h1[...], wr[...], preferred_element_type=jnp.float32)
         + b[...])
    h = jnp.maximum(h, 0.0)
    gids = lax.broadcasted_iota(jnp.int32, (R, G), 1)
    mask = (bt[...] == gids).astype(jnp.float32)
    ps = lax.dot_general(mask, h, (((0,), (0,)), ((), ())),
                         preferred_element_type=jnp.float32)
    pc = lax.dot_general(mask, jnp.ones((R, D), jnp.float32),
                         (((0,), (0,)), ((), ())),
                         preferred_element_type=jnp.float32)

    @pl.when(i == 0)
    def _():
        gsum[...] = jnp.zeros_like(gsum)
        gcnt[...] = jnp.zeros_like(gcnt)

    gsum[...] += ps
    gcnt[...] += pc

    @pl.when(i == NB - 1)
    def _():
        g = gsum[...] / jnp.maximum(gcnt[...], 1.0)
        out[...] = (jnp.dot(g, wlin[...], preferred_element_type=jnp.float32)
                    + blin[...])


_dense2 = pl.pallas_call(
    _dense2_body,
    grid=(NB,),
    in_specs=[
        pl.BlockSpec((R, D), lambda i: (i, 0)),
        pl.BlockSpec((R, D), lambda i: (i + NB, 0)),
        pl.BlockSpec((R, 1), lambda i: (i, 0)),
        pl.BlockSpec((R, D), lambda i: (i, 0)),
        pl.BlockSpec((D, D), lambda i: (0, 0)),
        pl.BlockSpec((D, D), lambda i: (0, 0)),
        pl.BlockSpec((1, D), lambda i: (0, 0)),
        pl.BlockSpec((R, 1), lambda i: (i, 0)),
        pl.BlockSpec((D, 1), lambda i: (0, 0)),
        pl.BlockSpec((1, 1), lambda i: (0, 0)),
    ],
    out_specs=pl.BlockSpec((G, 1), lambda i: (0, 0)),
    out_shape=jax.ShapeDtypeStruct((G, 1), jnp.float32),
    scratch_shapes=[pltpu.VMEM((G, D), jnp.float32),
                    pltpu.VMEM((G, D), jnp.float32)],
    compiler_params=pltpu.CompilerParams(
        dimension_semantics=("arbitrary",)),
)


def kernel(x, edge_index, batch, W1l, b1l, W1r, b1r, W2l, b2l, W2r, b2r,
           Wlin, blin):
    src = edge_index[0]
    dst = edge_index[1]
    z2d = jnp.zeros((CH, D), jnp.float32)
    o2d = jnp.ones((CH, D), jnp.float32)

    (ps1,) = _agg(x, src, dst, z2d)
    (cntp,) = _deg(dst, z2d, o2d)
    h1, rc = _dense1(ps1, ps1, cntp, cntp, x, W1l.T, W1r.T,
                     (b1l + b1r).reshape(1, D))
    (ps2,) = _agg(h1, src, dst, z2d)
    out = _dense2(ps2, ps2, rc, h1, W2l.T, W2r.T,
                  (b2l + b2r).reshape(1, D),
                  batch.reshape(N, 1), Wlin.T, blin.reshape(1, 1))
    return out.reshape(G)
